# Initial kernel scaffold; baseline (speedup 1.0000x reference)
import jax
import jax.numpy as jnp
from jax.experimental import pallas as pl


def kernel(x, edge_index, W1, att_src1, att_dst1, b1, W2, att_src2, att_dst2, b2):
    raise NotImplementedError("write your pallas kernel here")


# speedup vs baseline: 65.1686x; 65.1686x over previous
import functools

import jax
import jax.numpy as jnp
from jax import lax
from jax.experimental import pallas as pl
from jax.experimental.pallas import tpu as pltpu
from jax.experimental.pallas import tpu_sc as plsc

F32 = jnp.float32
I32 = jnp.int32

EPB = 128
NROW_CH = 128
NW = 32
NSUB = 16


def _mesh():
    return plsc.VectorSubcoreMesh(core_axis_name="c", subcore_axis_name="s",
                                  num_cores=2, num_subcores=NSUB)


def _c16(v):
    return jnp.full((16,), v, dtype=I32)


def _sc_layer1(n_nodes, n_edges):
    nblk = n_edges // EPB
    rows_per_tile = n_nodes // NSUB
    nch = rows_per_tile // NROW_CH

    @functools.partial(
        pl.kernel,
        out_type=jax.ShapeDtypeStruct((2, n_nodes, 16), F32),
        mesh=_mesh(),
        compiler_params=pltpu.CompilerParams(needs_layout_passes=False, use_tc_tiling_on_sc=False),
        scratch_types=[
            pltpu.VMEM((EPB,), I32),
            pltpu.VMEM((EPB,), I32),
            pltpu.VMEM((EPB, 16), F32),
            pltpu.VMEM((EPB, 16), F32),
            pltpu.VMEM((EPB, 16), F32),
            pltpu.VMEM((NROW_CH, 16), F32),
            pltpu.SemaphoreType.DMA,
            pltpu.SemaphoreType.DMA,
            pltpu.VMEM_SHARED((n_nodes, 16), F32),
        ],
    )
    def k(src_hbm, dst_hbm, xa_hbm, ad_hbm, gp_hbm,
          src_v, dst_v, rs_v, rd_v, orow_v, ch_v, sem1, sem2, g_sh):
        cid = lax.axis_index("c")
        sid = lax.axis_index("s")
        wid = cid * NSUB + sid
        lane = lax.iota(I32, 16)
        row0 = sid * rows_per_tile

        def _zb(i, _):
            ch_v[i, :] = jnp.zeros((16,), F32)
            return 0
        lax.fori_loop(0, NROW_CH, _zb, 0)

        def _zs(kk, _):
            pltpu.sync_copy(ch_v, g_sh.at[pl.ds(row0 + kk * NROW_CH, NROW_CH)])
            return 0
        lax.fori_loop(0, nch, _zs, 0)

        def _zp(j, _):
            r = j * 16 + lane
            zz = jnp.zeros((16,), F32)
            for col in (12, 13, 14, 15):
                plsc.store_scatter(orow_v, [r, _c16(col)], zz)
            return 0
        lax.fori_loop(0, EPB // 16, _zp, 0)

        plsc.subcore_barrier()

        def _blk(i, _):
            blk = i * NW + wid

            @pl.when(blk < nblk)
            def _():
                base = blk * EPB
                pltpu.sync_copy(src_hbm.at[pl.ds(base, EPB)], src_v)
                pltpu.sync_copy(dst_hbm.at[pl.ds(base, EPB)], dst_v)
                d1 = pltpu.async_copy(xa_hbm.at[src_v], rs_v, sem1)
                d2 = pltpu.async_copy(ad_hbm.at[dst_v], rd_v, sem2)
                d1.wait()
                d2.wait()

                def _cmp(j, _):
                    r = j * 16 + lane
                    as0 = plsc.load_gather(rs_v, [r, _c16(5)])
                    as1 = plsc.load_gather(rs_v, [r, _c16(6)])
                    ad0 = plsc.load_gather(rd_v, [r, _c16(0)])
                    ad1 = plsc.load_gather(rd_v, [r, _c16(1)])
                    e0 = as0 + ad0
                    e0 = jnp.where(e0 >= 0.0, e0, e0 * 0.2)
                    x0 = jnp.exp(e0)
                    e1 = as1 + ad1
                    e1 = jnp.where(e1 >= 0.0, e1, e1 * 0.2)
                    x1 = jnp.exp(e1)
                    for d in range(5):
                        xd = plsc.load_gather(rs_v, [r, _c16(d)])
                        plsc.store_scatter(orow_v, [r, _c16(d)], xd * x0)
                        plsc.store_scatter(orow_v, [r, _c16(5 + d)], xd * x1)
                    plsc.store_scatter(orow_v, [r, _c16(10)], x0)
                    plsc.store_scatter(orow_v, [r, _c16(11)], x1)
                    return 0
                lax.fori_loop(0, EPB // 16, _cmp, 0)

                pltpu.sync_copy(orow_v, g_sh.at[dst_v], add=True)
            return 0
        lax.fori_loop(0, (nblk + NW - 1) // NW, _blk, 0)

        plsc.subcore_barrier()

        def _fl(kk, _):
            r0 = row0 + kk * NROW_CH
            pltpu.sync_copy(g_sh.at[pl.ds(r0, NROW_CH)], ch_v)
            pltpu.sync_copy(ch_v, gp_hbm.at[cid].at[pl.ds(r0, NROW_CH)])
            return 0
        lax.fori_loop(0, nch, _fl, 0)

    return k


def _sc_layer2_num(n_nodes, n_edges):
    nblk = n_edges // EPB
    rows_per_tile = n_nodes // NSUB
    nch = rows_per_tile // NROW_CH

    @functools.partial(
        pl.kernel,
        out_type=(
            jax.ShapeDtypeStruct((n_edges,), F32),
            jax.ShapeDtypeStruct((2, n_nodes, 16), F32),
        ),
        mesh=_mesh(),
        compiler_params=pltpu.CompilerParams(needs_layout_passes=False, use_tc_tiling_on_sc=False),
        scratch_types=[
            pltpu.VMEM((EPB,), I32),
            pltpu.VMEM((EPB,), I32),
            pltpu.VMEM((EPB, 16), F32),
            pltpu.VMEM((EPB, 16), F32),
            pltpu.VMEM((EPB, 16), F32),
            pltpu.VMEM((EPB,), F32),
            pltpu.VMEM((NROW_CH, 16), F32),
            pltpu.SemaphoreType.DMA,
            pltpu.SemaphoreType.DMA,
            pltpu.VMEM_SHARED((n_nodes, 16), F32),
        ],
    )
    def k(src_hbm, dst_hbm, ar_hbm, ex_hbm, dp_hbm,
          src_v, dst_v, rs_v, rd_v, orow_v, ex_v, ch_v, sem1, sem2, den_sh):
        cid = lax.axis_index("c")
        sid = lax.axis_index("s")
        wid = cid * NSUB + sid
        lane = lax.iota(I32, 16)
        row0 = sid * rows_per_tile

        def _zb(i, _):
            ch_v[i, :] = jnp.zeros((16,), F32)
            return 0
        lax.fori_loop(0, NROW_CH, _zb, 0)

        def _zs(kk, _):
            pltpu.sync_copy(ch_v, den_sh.at[pl.ds(row0 + kk * NROW_CH, NROW_CH)])
            return 0
        lax.fori_loop(0, nch, _zs, 0)

        def _zp(j, _):
            r = j * 16 + lane
            zz = jnp.zeros((16,), F32)
            for col in range(1, 16):
                plsc.store_scatter(orow_v, [r, _c16(col)], zz)
            return 0
        lax.fori_loop(0, EPB // 16, _zp, 0)

        plsc.subcore_barrier()

        def _blk(i, _):
            blk = i * NW + wid

            @pl.when(blk < nblk)
            def _():
                base = blk * EPB
                pltpu.sync_copy(src_hbm.at[pl.ds(base, EPB)], src_v)
                pltpu.sync_copy(dst_hbm.at[pl.ds(base, EPB)], dst_v)
                d1 = pltpu.async_copy(ar_hbm.at[src_v], rs_v, sem1)
                d2 = pltpu.async_copy(ar_hbm.at[dst_v], rd_v, sem2)
                d1.wait()
                d2.wait()

                def _cmp(j, _):
                    r = j * 16 + lane
                    a_s = plsc.load_gather(rs_v, [r, _c16(0)])
                    a_d = plsc.load_gather(rd_v, [r, _c16(1)])
                    e = a_s + a_d
                    e = jnp.where(e >= 0.0, e, e * 0.2)
                    xv = jnp.exp(e)
                    ex_v[pl.ds(j * 16, 16)] = xv
                    plsc.store_scatter(orow_v, [r, _c16(0)], xv)
                    return 0
                lax.fori_loop(0, EPB // 16, _cmp, 0)

                pltpu.sync_copy(ex_v, ex_hbm.at[pl.ds(base, EPB)])
                pltpu.sync_copy(orow_v, den_sh.at[dst_v], add=True)
            return 0
        lax.fori_loop(0, (nblk + NW - 1) // NW, _blk, 0)

        plsc.subcore_barrier()

        def _fl(kk, _):
            r0 = row0 + kk * NROW_CH
            pltpu.sync_copy(den_sh.at[pl.ds(r0, NROW_CH)], ch_v)
            pltpu.sync_copy(ch_v, dp_hbm.at[cid].at[pl.ds(r0, NROW_CH)])
            return 0
        lax.fori_loop(0, nch, _fl, 0)

    return k


def _sc_layer2_agg(n_nodes, n_edges):
    nblk = n_edges // EPB
    rows_per_tile = n_nodes // NSUB
    nch = rows_per_tile // NROW_CH

    @functools.partial(
        pl.kernel,
        out_type=jax.ShapeDtypeStruct((2, n_nodes, 16), F32),
        mesh=_mesh(),
        compiler_params=pltpu.CompilerParams(needs_layout_passes=False, use_tc_tiling_on_sc=False),
        scratch_types=[
            pltpu.VMEM((EPB,), I32),
            pltpu.VMEM((EPB,), I32),
            pltpu.VMEM((EPB, 16), F32),
            pltpu.VMEM((EPB, 16), F32),
            pltpu.VMEM((EPB,), F32),
            pltpu.VMEM((NROW_CH, 16), F32),
            pltpu.SemaphoreType.DMA,
            pltpu.VMEM_SHARED((n_nodes, 16), F32),
        ],
    )
    def k(src_hbm, dst_hbm, ex_hbm, h2_hbm, ap_hbm,
          src_v, dst_v, rs_v, orow_v, ex_v, ch_v, sem1, acc_sh):
        cid = lax.axis_index("c")
        sid = lax.axis_index("s")
        wid = cid * NSUB + sid
        lane = lax.iota(I32, 16)
        row0 = sid * rows_per_tile

        def _zb(i, _):
            ch_v[i, :] = jnp.zeros((16,), F32)
            return 0
        lax.fori_loop(0, NROW_CH, _zb, 0)

        def _zs(kk, _):
            pltpu.sync_copy(ch_v, acc_sh.at[pl.ds(row0 + kk * NROW_CH, NROW_CH)])
            return 0
        lax.fori_loop(0, nch, _zs, 0)

        plsc.subcore_barrier()

        def _blk(i, _):
            blk = i * NW + wid

            @pl.when(blk < nblk)
            def _():
                base = blk * EPB
                pltpu.sync_copy(src_hbm.at[pl.ds(base, EPB)], src_v)
                pltpu.sync_copy(dst_hbm.at[pl.ds(base, EPB)], dst_v)
                pltpu.sync_copy(ex_hbm.at[pl.ds(base, EPB)], ex_v)
                d1 = pltpu.async_copy(h2_hbm.at[src_v], rs_v, sem1)
                d1.wait()

                def _cmp(j, _):
                    r = j * 16 + lane
                    xv = ex_v[pl.ds(j * 16, 16)]
                    for col in range(16):
                        v = plsc.load_gather(rs_v, [r, _c16(col)])
                        plsc.store_scatter(orow_v, [r, _c16(col)], v * xv)
                    return 0
                lax.fori_loop(0, EPB // 16, _cmp, 0)

                pltpu.sync_copy(orow_v, acc_sh.at[dst_v], add=True)
            return 0
        lax.fori_loop(0, (nblk + NW - 1) // NW, _blk, 0)

        plsc.subcore_barrier()

        def _fl(kk, _):
            r0 = row0 + kk * NROW_CH
            pltpu.sync_copy(acc_sh.at[pl.ds(r0, NROW_CH)], ch_v)
            pltpu.sync_copy(ch_v, ap_hbm.at[cid].at[pl.ds(r0, NROW_CH)])
            return 0
        lax.fori_loop(0, nch, _fl, 0)

    return k


_TC_BLK = 2048


def _tc_prep1(n_nodes):
    def body(x_ref, was_ref, wad_ref, xa_ref, ar_ref):
        xb = x_ref[...]
        a_s = jnp.dot(xb, was_ref[...], preferred_element_type=F32)
        a_d = jnp.dot(xb, wad_ref[...], preferred_element_type=F32)
        z9 = jnp.zeros((xb.shape[0], 9), F32)
        z14 = jnp.zeros((xb.shape[0], 14), F32)
        xa_ref[...] = jnp.concatenate([xb, a_s, z9], axis=1)
        ar_ref[...] = jnp.concatenate([a_d, z14], axis=1)

    return pl.pallas_call(
        body,
        grid=(n_nodes // _TC_BLK,),
        in_specs=[
            pl.BlockSpec((_TC_BLK, 5), lambda i: (i, 0)),
            pl.BlockSpec((5, 2), lambda i: (0, 0)),
            pl.BlockSpec((5, 2), lambda i: (0, 0)),
        ],
        out_specs=[
            pl.BlockSpec((_TC_BLK, 16), lambda i: (i, 0)),
            pl.BlockSpec((_TC_BLK, 16), lambda i: (i, 0)),
        ],
        out_shape=[
            jax.ShapeDtypeStruct((n_nodes, 16), F32),
            jax.ShapeDtypeStruct((n_nodes, 16), F32),
        ],
    )


def _tc_mid(n_nodes):
    def body(gp_ref, w1_ref, b1_ref, w2_ref, att2_ref, h2_ref, ar_ref):
        g = gp_ref[0] + gp_ref[1]
        d0 = g[:, 10:11] + 1e-16
        d1 = g[:, 11:12] + 1e-16
        w1 = w1_ref[...]
        h0 = jnp.dot(g[:, 0:5], w1[:, :32], preferred_element_type=F32) / d0
        h1 = jnp.dot(g[:, 5:10], w1[:, 32:], preferred_element_type=F32) / d1
        h2in = jnp.maximum(jnp.concatenate([h0, h1], axis=1) + b1_ref[...], 0.0)
        h2 = jnp.dot(h2in, w2_ref[...], preferred_element_type=F32)
        att2 = att2_ref[...]
        a_s = jnp.sum(h2 * att2[0:1, :], axis=1, keepdims=True)
        a_d = jnp.sum(h2 * att2[1:2, :], axis=1, keepdims=True)
        z14 = jnp.zeros((h2.shape[0], 14), F32)
        h2_ref[...] = h2
        ar_ref[...] = jnp.concatenate([a_s, a_d, z14], axis=1)

    return pl.pallas_call(
        body,
        grid=(n_nodes // _TC_BLK,),
        in_specs=[
            pl.BlockSpec((2, _TC_BLK, 16), lambda i: (0, i, 0)),
            pl.BlockSpec((5, 64), lambda i: (0, 0)),
            pl.BlockSpec((1, 64), lambda i: (0, 0)),
            pl.BlockSpec((64, 16), lambda i: (0, 0)),
            pl.BlockSpec((2, 16), lambda i: (0, 0)),
        ],
        out_specs=[
            pl.BlockSpec((_TC_BLK, 16), lambda i: (i, 0)),
            pl.BlockSpec((_TC_BLK, 16), lambda i: (i, 0)),
        ],
        out_shape=[
            jax.ShapeDtypeStruct((n_nodes, 16), F32),
            jax.ShapeDtypeStruct((n_nodes, 16), F32),
        ],
    )


def _tc_final(n_nodes):
    def body(ap_ref, dp_ref, b2_ref, out_ref):
        acc = ap_ref[0] + ap_ref[1]
        den = dp_ref[0][:, 0:1] + dp_ref[1][:, 0:1] + 1e-16
        out_ref[...] = acc / den + b2_ref[...]

    return pl.pallas_call(
        body,
        grid=(n_nodes // _TC_BLK,),
        in_specs=[
            pl.BlockSpec((2, _TC_BLK, 16), lambda i: (0, i, 0)),
            pl.BlockSpec((2, _TC_BLK, 16), lambda i: (0, i, 0)),
            pl.BlockSpec((1, 16), lambda i: (0, 0)),
        ],
        out_specs=pl.BlockSpec((_TC_BLK, 16), lambda i: (i, 0)),
        out_shape=jax.ShapeDtypeStruct((n_nodes, 16), F32),
    )


@jax.jit
def kernel(x, edge_index, W1, att_src1, att_dst1, b1, W2, att_src2, att_dst2, b2):
    n_nodes = x.shape[0]
    n_edges = edge_index.shape[1]
    n_pad = -(-n_nodes // (NSUB * NROW_CH)) * (NSUB * NROW_CH)
    src = edge_index[0].astype(I32)
    dst = edge_index[1].astype(I32)
    xp = jnp.pad(x, ((0, n_pad - n_nodes), (0, 0)))

    w3 = W1.reshape(x.shape[1], att_src1.shape[0], att_src1.shape[1])
    w_as1 = (w3 * att_src1[None]).sum(-1)
    w_ad1 = (w3 * att_dst1[None]).sum(-1)

    xa, arow1 = _tc_prep1(n_pad)(xp, w_as1, w_ad1)
    gp = _sc_layer1(n_pad, n_edges)(src, dst, xa, arow1)
    att2 = jnp.concatenate([att_src2, att_dst2], axis=0)
    h2row, arow2 = _tc_mid(n_pad)(gp, W1, b1.reshape(1, -1), W2, att2)
    ex2, denp = _sc_layer2_num(n_pad, n_edges)(src, dst, arow2)
    accp = _sc_layer2_agg(n_pad, n_edges)(src, dst, ex2, h2row)
    out = _tc_final(n_pad)(accp, denp, b2.reshape(1, -1))
    return out[:n_nodes]


# speedup vs baseline: 87.8433x; 1.3479x over previous
import functools

import jax
import jax.numpy as jnp
from jax import lax
from jax.experimental import pallas as pl
from jax.experimental.pallas import tpu as pltpu
from jax.experimental.pallas import tpu_sc as plsc

F32 = jnp.float32
I32 = jnp.int32

EPB = 128
NROW_CH = 128
NW = 32
NSUB = 16

_SC_PARAMS = pltpu.CompilerParams(needs_layout_passes=False,
                                  use_tc_tiling_on_sc=False)


def _mesh():
    return plsc.VectorSubcoreMesh(core_axis_name="c", subcore_axis_name="s",
                                  num_cores=2, num_subcores=NSUB)


def _c16(v):
    return jnp.full((16,), v, dtype=I32)


def _zero_spmem_stripe(ch_v, sh_ref, row0, nch):
    def _zb(i, _):
        ch_v[i, :] = jnp.zeros((16,), F32)
        return 0
    lax.fori_loop(0, NROW_CH, _zb, 0)

    def _zs(kk, _):
        pltpu.sync_copy(ch_v, sh_ref.at[pl.ds(row0 + kk * NROW_CH, NROW_CH)])
        return 0
    lax.fori_loop(0, nch, _zs, 0)


def _flush_spmem_stripe(ch_v, sh_ref, out_hbm, cid, row0, nch):
    def _fl(kk, _):
        r0 = row0 + kk * NROW_CH
        pltpu.sync_copy(sh_ref.at[pl.ds(r0, NROW_CH)], ch_v)
        pltpu.sync_copy(ch_v, out_hbm.at[cid].at[pl.ds(r0, NROW_CH)])
        return 0
    lax.fori_loop(0, nch, _fl, 0)


def _sc_layer1(n_nodes, n_edges):
    nblk = n_edges // EPB
    rows_per_tile = n_nodes // NSUB
    nch = rows_per_tile // NROW_CH
    tmax = -(-nblk // NW)

    @functools.partial(
        pl.kernel,
        out_type=jax.ShapeDtypeStruct((2, n_nodes, 16), F32),
        mesh=_mesh(),
        compiler_params=_SC_PARAMS,
        scratch_types=[
            pltpu.VMEM((2, 2, EPB), I32),
            pltpu.VMEM((EPB, 8), F32),
            pltpu.VMEM((EPB, 8), F32),
            pltpu.VMEM((EPB, 8), F32),
            pltpu.VMEM((EPB, 8), F32),
            pltpu.VMEM((EPB, 16), F32),
            pltpu.VMEM((EPB, 16), F32),
            pltpu.VMEM((NROW_CH, 16), F32),
            pltpu.SemaphoreType.DMA,
            pltpu.SemaphoreType.DMA,
            pltpu.SemaphoreType.DMA,
            pltpu.SemaphoreType.DMA,
            pltpu.VMEM_SHARED((n_nodes, 16), F32),
        ],
    )
    def k(ei_hbm, xa_hbm, ad_hbm, gp_hbm,
          sd_v, rs0, rs1, rd0, rd1, or0, or1, ch_v,
          ss0, ss1, sdm0, sdm1, g_sh):
        cid = lax.axis_index("c")
        sid = lax.axis_index("s")
        wid = cid * NSUB + sid
        lane = lax.iota(I32, 16)
        row0 = sid * rows_per_tile
        rs = (rs0, rs1)
        rd = (rd0, rd1)
        orow = (or0, or1)
        sems_s = (ss0, ss1)
        sems_d = (sdm0, sdm1)

        _zero_spmem_stripe(ch_v, g_sh, row0, nch)

        def _zp(j, _):
            r = j * 16 + lane
            zz = jnp.zeros((16,), F32)
            for col in (12, 13, 14, 15):
                plsc.store_scatter(or0, [r, _c16(col)], zz)
                plsc.store_scatter(or1, [r, _c16(col)], zz)
            return 0
        lax.fori_loop(0, EPB // 16, _zp, 0)

        plsc.subcore_barrier()

        def _prefetch(t, b):
            @pl.when(t * NW + wid < nblk)
            def _():
                base = (t * NW + wid) * EPB
                pltpu.sync_copy(ei_hbm.at[:, pl.ds(base, EPB)], sd_v.at[b])
                pltpu.async_copy(xa_hbm.at[sd_v.at[b, 0]], rs[b], sems_s[b])
                pltpu.async_copy(ad_hbm.at[sd_v.at[b, 1]], rd[b], sems_d[b])

        def _process(t, b):
            @pl.when(t * NW + wid < nblk)
            def _():
                pltpu.make_async_copy(xa_hbm.at[sd_v.at[b, 0]], rs[b],
                                      sems_s[b]).wait()
                pltpu.make_async_copy(ad_hbm.at[sd_v.at[b, 1]], rd[b],
                                      sems_d[b]).wait()

                def _cmp(j, _):
                    r = j * 16 + lane
                    as0 = plsc.load_gather(rs[b], [r, _c16(5)])
                    as1 = plsc.load_gather(rs[b], [r, _c16(6)])
                    ad0 = plsc.load_gather(rd[b], [r, _c16(0)])
                    ad1 = plsc.load_gather(rd[b], [r, _c16(1)])
                    e0 = as0 + ad0
                    e0 = jnp.where(e0 >= 0.0, e0, e0 * 0.2)
                    x0 = jnp.exp(e0)
                    e1 = as1 + ad1
                    e1 = jnp.where(e1 >= 0.0, e1, e1 * 0.2)
                    x1 = jnp.exp(e1)
                    for d in range(5):
                        xd = plsc.load_gather(rs[b], [r, _c16(d)])
                        plsc.store_scatter(orow[b], [r, _c16(d)], xd * x0)
                        plsc.store_scatter(orow[b], [r, _c16(5 + d)], xd * x1)
                    plsc.store_scatter(orow[b], [r, _c16(10)], x0)
                    plsc.store_scatter(orow[b], [r, _c16(11)], x1)
                    return 0
                lax.fori_loop(0, EPB // 16, _cmp, 0)

                pltpu.sync_copy(orow[b], g_sh.at[sd_v.at[b, 1]], add=True)

        _prefetch(0, 0)

        def _pair(p, _):
            for b in range(2):
                t = 2 * p + b
                _prefetch(t + 1, 1 - b)
                _process(t, b)
            return 0
        lax.fori_loop(0, (tmax + 1) // 2, _pair, 0)

        plsc.subcore_barrier()
        _flush_spmem_stripe(ch_v, g_sh, gp_hbm, cid, row0, nch)

    return k


def _sc_layer2(n_nodes, n_edges):
    nblk = n_edges // EPB
    half = n_nodes // 2
    hp = -(-half + NSUB * NROW_CH - 1) // (NSUB * NROW_CH) * (NSUB * NROW_CH)
    hp = ((half // (NSUB * NROW_CH)) + 1) * (NSUB * NROW_CH)
    rows_per_tile = hp // NSUB
    nch = rows_per_tile // NROW_CH
    tmax = -(-nblk // NSUB)

    @functools.partial(
        pl.kernel,
        out_type=(
            jax.ShapeDtypeStruct((2, hp, 16), F32),
            jax.ShapeDtypeStruct((2, hp, 8), F32),
        ),
        mesh=_mesh(),
        compiler_params=_SC_PARAMS,
        scratch_types=[
            pltpu.VMEM((2, 2, EPB), I32),
            pltpu.VMEM((2, EPB), I32),
            pltpu.VMEM((EPB, 16), F32),
            pltpu.VMEM((EPB, 16), F32),
            pltpu.VMEM((EPB, 8), F32),
            pltpu.VMEM((EPB, 8), F32),
            pltpu.VMEM((EPB, 8), F32),
            pltpu.VMEM((EPB, 8), F32),
            pltpu.VMEM((EPB, 16), F32),
            pltpu.VMEM((EPB, 16), F32),
            pltpu.VMEM((EPB, 8), F32),
            pltpu.VMEM((EPB, 8), F32),
            pltpu.VMEM((NROW_CH, 16), F32),
            pltpu.VMEM((NROW_CH, 8), F32),
            pltpu.SemaphoreType.DMA,
            pltpu.SemaphoreType.DMA,
            pltpu.SemaphoreType.DMA,
            pltpu.SemaphoreType.DMA,
            pltpu.SemaphoreType.DMA,
            pltpu.SemaphoreType.DMA,
            pltpu.VMEM_SHARED((hp, 16), F32),
            pltpu.VMEM_SHARED((hp, 8), F32),
        ],
    )
    def k(ei_hbm, h2_hbm, ar_hbm, ap_hbm, dp_hbm,
          sd_v, li_v, rh0, rh1, rs0, rs1, rd0, rd1, oa0, oa1, od0, od1,
          ch_v, chd_v, sh0, sh1, ss0, ss1, sdm0, sdm1, acc_sh, den_sh):
        cid = lax.axis_index("c")
        sid = lax.axis_index("s")
        lane = lax.iota(I32, 16)
        row0 = sid * rows_per_tile
        base = cid * half
        rh = (rh0, rh1)
        rs = (rs0, rs1)
        rd = (rd0, rd1)
        oa = (oa0, oa1)
        od = (od0, od1)
        sems_h = (sh0, sh1)
        sems_s = (ss0, ss1)
        sems_d = (sdm0, sdm1)

        _zero_spmem_stripe(ch_v, acc_sh, row0, nch)

        def _zbd2(i, _):
            def _zc(j, _):
                plsc.store_scatter(chd_v, [jnp.full((16,), i, I32),
                                           (j * 16 + lane) % 8],
                                   jnp.zeros((16,), F32))
                return 0
            lax.fori_loop(0, 1, _zc, 0)
            return 0
        lax.fori_loop(0, NROW_CH, _zbd2, 0)

        def _zsd(kk, _):
            pltpu.sync_copy(chd_v, den_sh.at[pl.ds(row0 + kk * NROW_CH,
                                                   NROW_CH)])
            return 0
        lax.fori_loop(0, nch, _zsd, 0)

        def _zp(j, _):
            r = j * 16 + lane
            zz = jnp.zeros((16,), F32)
            for col in range(1, 8):
                plsc.store_scatter(od0, [r, _c16(col)], zz)
                plsc.store_scatter(od1, [r, _c16(col)], zz)
            return 0
        lax.fori_loop(0, EPB // 16, _zp, 0)

        plsc.subcore_barrier()

        def _prefetch(t, b):
            @pl.when(t * NSUB + sid < nblk)
            def _():
                bs = (t * NSUB + sid) * EPB
                pltpu.sync_copy(ei_hbm.at[:, pl.ds(bs, EPB)], sd_v.at[b])
                pltpu.async_copy(h2_hbm.at[sd_v.at[b, 0]], rh[b], sems_h[b])
                pltpu.async_copy(ar_hbm.at[sd_v.at[b, 0]], rs[b], sems_s[b])
                pltpu.async_copy(ar_hbm.at[sd_v.at[b, 1]], rd[b], sems_d[b])

        def _process(t, b):
            @pl.when(t * NSUB + sid < nblk)
            def _():
                pltpu.make_async_copy(h2_hbm.at[sd_v.at[b, 0]], rh[b],
                                      sems_h[b]).wait()
                pltpu.make_async_copy(ar_hbm.at[sd_v.at[b, 0]], rs[b],
                                      sems_s[b]).wait()
                pltpu.make_async_copy(ar_hbm.at[sd_v.at[b, 1]], rd[b],
                                      sems_d[b]).wait()

                def _cmp(j, _):
                    r = j * 16 + lane
                    a_s = plsc.load_gather(rs[b], [r, _c16(0)])
                    a_d = plsc.load_gather(rd[b], [r, _c16(1)])
                    e = a_s + a_d
                    e = jnp.where(e >= 0.0, e, e * 0.2)
                    xv = jnp.exp(e)
                    for d in range(16):
                        v = plsc.load_gather(rh[b], [r, _c16(d)])
                        plsc.store_scatter(oa[b], [r, _c16(d)], v * xv)
                    plsc.store_scatter(od[b], [r, _c16(0)], xv)
                    dsts = plsc.load_gather(sd_v.at[b, 1], [r])
                    du = dsts - base
                    inr = (du >= 0) & (du < half)
                    li = jnp.where(inr, du, half)
                    li_v[b, pl.ds(j * 16, 16)] = li
                    return 0
                lax.fori_loop(0, EPB // 16, _cmp, 0)

                pltpu.sync_copy(oa[b], acc_sh.at[li_v.at[b]], add=True)
                pltpu.sync_copy(od[b], den_sh.at[li_v.at[b]], add=True)

        _prefetch(0, 0)

        def _pair(p, _):
            for b in range(2):
                t = 2 * p + b
                _prefetch(t + 1, 1 - b)
                _process(t, b)
            return 0
        lax.fori_loop(0, (tmax + 1) // 2, _pair, 0)

        plsc.subcore_barrier()
        _flush_spmem_stripe(ch_v, acc_sh, ap_hbm, cid, row0, nch)

        def _fld(kk, _):
            r0 = row0 + kk * NROW_CH
            pltpu.sync_copy(den_sh.at[pl.ds(r0, NROW_CH)], chd_v)
            pltpu.sync_copy(chd_v, dp_hbm.at[cid].at[pl.ds(r0, NROW_CH)])
            return 0
        lax.fori_loop(0, nch, _fld, 0)

    return k, hp


_TC_BLK = 2048


def _tc_prep1(n_nodes):
    def body(x_ref, was_ref, wad_ref, xa_ref, ar_ref):
        xb = x_ref[...]
        a_s = jnp.dot(xb, was_ref[...], preferred_element_type=F32)
        a_d = jnp.dot(xb, wad_ref[...], preferred_element_type=F32)
        z1 = jnp.zeros((xb.shape[0], 1), F32)
        z6 = jnp.zeros((xb.shape[0], 6), F32)
        xa_ref[...] = jnp.concatenate([xb, a_s, z1], axis=1)
        ar_ref[...] = jnp.concatenate([a_d, z6], axis=1)

    return pl.pallas_call(
        body,
        grid=(n_nodes // _TC_BLK,),
        in_specs=[
            pl.BlockSpec((_TC_BLK, 5), lambda i: (i, 0)),
            pl.BlockSpec((5, 2), lambda i: (0, 0)),
            pl.BlockSpec((5, 2), lambda i: (0, 0)),
        ],
        out_specs=[
            pl.BlockSpec((_TC_BLK, 8), lambda i: (i, 0)),
            pl.BlockSpec((_TC_BLK, 8), lambda i: (i, 0)),
        ],
        out_shape=[
            jax.ShapeDtypeStruct((n_nodes, 8), F32),
            jax.ShapeDtypeStruct((n_nodes, 8), F32),
        ],
    )


def _tc_mid(n_nodes):
    def body(gp_ref, w1_ref, b1_ref, w2_ref, att2_ref, h2_ref, ar_ref):
        g = gp_ref[0] + gp_ref[1]
        d0 = g[:, 10:11] + 1e-16
        d1 = g[:, 11:12] + 1e-16
        w1 = w1_ref[...]
        h0 = jnp.dot(g[:, 0:5], w1[:, :32], preferred_element_type=F32) / d0
        h1 = jnp.dot(g[:, 5:10], w1[:, 32:], preferred_element_type=F32) / d1
        h2in = jnp.maximum(jnp.concatenate([h0, h1], axis=1) + b1_ref[...], 0.0)
        h2 = jnp.dot(h2in, w2_ref[...], preferred_element_type=F32)
        att2 = att2_ref[...]
        a_s = jnp.sum(h2 * att2[0:1, :], axis=1, keepdims=True)
        a_d = jnp.sum(h2 * att2[1:2, :], axis=1, keepdims=True)
        z6 = jnp.zeros((h2.shape[0], 6), F32)
        h2_ref[...] = h2
        ar_ref[...] = jnp.concatenate([a_s, a_d, z6], axis=1)

    return pl.pallas_call(
        body,
        grid=(n_nodes // _TC_BLK,),
        in_specs=[
            pl.BlockSpec((2, _TC_BLK, 16), lambda i: (0, i, 0)),
            pl.BlockSpec((5, 64), lambda i: (0, 0)),
            pl.BlockSpec((1, 64), lambda i: (0, 0)),
            pl.BlockSpec((64, 16), lambda i: (0, 0)),
            pl.BlockSpec((2, 16), lambda i: (0, 0)),
        ],
        out_specs=[
            pl.BlockSpec((_TC_BLK, 16), lambda i: (i, 0)),
            pl.BlockSpec((_TC_BLK, 8), lambda i: (i, 0)),
        ],
        out_shape=[
            jax.ShapeDtypeStruct((n_nodes, 16), F32),
            jax.ShapeDtypeStruct((n_nodes, 8), F32),
        ],
    )


def _tc_final(n_nodes):
    def body(acc_ref, den_ref, b2_ref, out_ref):
        den = den_ref[:, 0:1] + 1e-16
        out_ref[...] = acc_ref[...] / den + b2_ref[...]

    return pl.pallas_call(
        body,
        grid=(n_nodes // _TC_BLK,),
        in_specs=[
            pl.BlockSpec((_TC_BLK, 16), lambda i: (i, 0)),
            pl.BlockSpec((_TC_BLK, 8), lambda i: (i, 0)),
            pl.BlockSpec((1, 16), lambda i: (0, 0)),
        ],
        out_specs=pl.BlockSpec((_TC_BLK, 16), lambda i: (i, 0)),
        out_shape=jax.ShapeDtypeStruct((n_nodes, 16), F32),
    )


@jax.jit
def kernel(x, edge_index, W1, att_src1, att_dst1, b1, W2, att_src2, att_dst2, b2):
    n_nodes = x.shape[0]
    n_edges = edge_index.shape[1]
    n_pad = -(-n_nodes // (NSUB * NROW_CH)) * (NSUB * NROW_CH)
    ei = edge_index.astype(I32)
    xp = jnp.pad(x, ((0, n_pad - n_nodes), (0, 0)))

    w3 = W1.reshape(x.shape[1], att_src1.shape[0], att_src1.shape[1])
    w_as1 = (w3 * att_src1[None]).sum(-1)
    w_ad1 = (w3 * att_dst1[None]).sum(-1)

    xa, arow1 = _tc_prep1(n_pad)(xp, w_as1, w_ad1)
    gp = _sc_layer1(n_pad, n_edges)(ei, xa, arow1)
    att2 = jnp.concatenate([att_src2, att_dst2], axis=0)
    h2row, arow2 = _tc_mid(n_pad)(gp, W1, b1.reshape(1, -1), W2, att2)
    l2, _hp = _sc_layer2(n_pad, n_edges)
    accp, denp = l2(ei, h2row, arow2)
    half = n_pad // 2
    acc_full = jnp.concatenate([accp[0, :half], accp[1, :half]], axis=0)
    den_full = jnp.concatenate([denp[0, :half], denp[1, :half]], axis=0)
    out = _tc_final(n_pad)(acc_full, den_full, b2.reshape(1, -1))
    return out[:n_nodes]


# speedup vs baseline: 103.0889x; 1.1736x over previous
import functools

import jax
import jax.numpy as jnp
from jax import lax
from jax.experimental import pallas as pl
from jax.experimental.pallas import tpu as pltpu
from jax.experimental.pallas import tpu_sc as plsc

F32 = jnp.float32
I32 = jnp.int32

EPB = 128
NROW_CH = 128
NW = 32
NSUB = 16

_SC_PARAMS = pltpu.CompilerParams(needs_layout_passes=False,
                                  use_tc_tiling_on_sc=False)


def _mesh():
    return plsc.VectorSubcoreMesh(core_axis_name="c", subcore_axis_name="s",
                                  num_cores=2, num_subcores=NSUB)


def _c16(v):
    return jnp.full((16,), v, dtype=I32)


def _zero_spmem_stripe(ch_v, sh_ref, row0, nch):
    def _zb(i, _):
        ch_v[i, :] = jnp.zeros((16,), F32)
        return 0
    lax.fori_loop(0, NROW_CH, _zb, 0)

    def _zs(kk, _):
        pltpu.sync_copy(ch_v, sh_ref.at[pl.ds(row0 + kk * NROW_CH, NROW_CH)])
        return 0
    lax.fori_loop(0, nch, _zs, 0)


def _flush_spmem_stripe(ch_v, sh_ref, out_hbm, cid, row0, nch):
    def _fl(kk, _):
        r0 = row0 + kk * NROW_CH
        pltpu.sync_copy(sh_ref.at[pl.ds(r0, NROW_CH)], ch_v)
        pltpu.sync_copy(ch_v, out_hbm.at[cid].at[pl.ds(r0, NROW_CH)])
        return 0
    lax.fori_loop(0, nch, _fl, 0)


def _sc_layer1(n_nodes, n_edges):
    nblk = n_edges // EPB
    rows_per_tile = n_nodes // NSUB
    nch = rows_per_tile // NROW_CH
    tmax = -(-nblk // NW)

    @functools.partial(
        pl.kernel,
        out_type=jax.ShapeDtypeStruct((2, n_nodes, 16), F32),
        mesh=_mesh(),
        compiler_params=_SC_PARAMS,
        scratch_types=[
            pltpu.VMEM((2, 2, EPB), I32),
            pltpu.VMEM((EPB, 8), F32),
            pltpu.VMEM((EPB, 8), F32),
            pltpu.VMEM((EPB, 8), F32),
            pltpu.VMEM((EPB, 8), F32),
            pltpu.VMEM((EPB, 16), F32),
            pltpu.VMEM((EPB, 16), F32),
            pltpu.VMEM((NROW_CH, 16), F32),
            pltpu.SemaphoreType.DMA,
            pltpu.SemaphoreType.DMA,
            pltpu.SemaphoreType.DMA,
            pltpu.SemaphoreType.DMA,
            pltpu.VMEM_SHARED((n_nodes, 16), F32),
        ],
    )
    def k(ei_hbm, xa_hbm, ad_hbm, gp_hbm,
          sd_v, rs0, rs1, rd0, rd1, or0, or1, ch_v,
          ss0, ss1, sdm0, sdm1, g_sh):
        cid = lax.axis_index("c")
        sid = lax.axis_index("s")
        wid = cid * NSUB + sid
        lane = lax.iota(I32, 16)
        row0 = sid * rows_per_tile
        rs = (rs0, rs1)
        rd = (rd0, rd1)
        orow = (or0, or1)
        sems_s = (ss0, ss1)
        sems_d = (sdm0, sdm1)

        _zero_spmem_stripe(ch_v, g_sh, row0, nch)

        def _zp(j, _):
            r = j * 16 + lane
            zz = jnp.zeros((16,), F32)
            for col in (12, 13, 14, 15):
                plsc.store_scatter(or0, [r, _c16(col)], zz)
                plsc.store_scatter(or1, [r, _c16(col)], zz)
            return 0
        lax.fori_loop(0, EPB // 16, _zp, 0)

        plsc.subcore_barrier()

        def _prefetch(t, b):
            @pl.when(t * NW + wid < nblk)
            def _():
                base = (t * NW + wid) * EPB
                pltpu.sync_copy(ei_hbm.at[:, pl.ds(base, EPB)], sd_v.at[b])
                pltpu.async_copy(xa_hbm.at[sd_v.at[b, 0]], rs[b], sems_s[b])
                pltpu.async_copy(ad_hbm.at[sd_v.at[b, 1]], rd[b], sems_d[b])

        def _process(t, b):
            @pl.when(t * NW + wid < nblk)
            def _():
                pltpu.make_async_copy(xa_hbm.at[sd_v.at[b, 0]], rs[b],
                                      sems_s[b]).wait()
                pltpu.make_async_copy(ad_hbm.at[sd_v.at[b, 1]], rd[b],
                                      sems_d[b]).wait()

                def _cmp(j, _):
                    r = j * 16 + lane
                    as0 = plsc.load_gather(rs[b], [r, _c16(5)])
                    as1 = plsc.load_gather(rs[b], [r, _c16(6)])
                    ad0 = plsc.load_gather(rd[b], [r, _c16(0)])
                    ad1 = plsc.load_gather(rd[b], [r, _c16(1)])
                    e0 = as0 + ad0
                    e0 = jnp.where(e0 >= 0.0, e0, e0 * 0.2)
                    x0 = jnp.exp(e0)
                    e1 = as1 + ad1
                    e1 = jnp.where(e1 >= 0.0, e1, e1 * 0.2)
                    x1 = jnp.exp(e1)
                    for d in range(5):
                        xd = plsc.load_gather(rs[b], [r, _c16(d)])
                        plsc.store_scatter(orow[b], [r, _c16(d)], xd * x0)
                        plsc.store_scatter(orow[b], [r, _c16(5 + d)], xd * x1)
                    plsc.store_scatter(orow[b], [r, _c16(10)], x0)
                    plsc.store_scatter(orow[b], [r, _c16(11)], x1)
                    return 0
                lax.fori_loop(0, EPB // 16, _cmp, 0)

                pltpu.sync_copy(orow[b], g_sh.at[sd_v.at[b, 1]], add=True)

        _prefetch(0, 0)

        def _pair(p, _):
            for b in range(2):
                t = 2 * p + b
                _prefetch(t + 1, 1 - b)
                _process(t, b)
            return 0
        lax.fori_loop(0, (tmax + 1) // 2, _pair, 0)

        plsc.subcore_barrier()
        _flush_spmem_stripe(ch_v, g_sh, gp_hbm, cid, row0, nch)

    return k


def _sc_layer2(n_nodes, n_edges):
    nblk = n_edges // EPB
    half = n_nodes // 2
    hp = ((half // (NSUB * NROW_CH)) + 1) * (NSUB * NROW_CH)
    rows_per_tile = hp // NSUB
    nch = rows_per_tile // NROW_CH

    @functools.partial(
        pl.kernel,
        out_type=jax.ShapeDtypeStruct((2, hp, 24), F32),
        mesh=_mesh(),
        compiler_params=_SC_PARAMS,
        scratch_types=[
            pltpu.VMEM((2, 2, EPB), I32),
            pltpu.VMEM((2, EPB), I32),
            pltpu.VMEM((EPB, 24), F32),
            pltpu.VMEM((EPB, 24), F32),
            pltpu.VMEM((EPB, 8), F32),
            pltpu.VMEM((EPB, 8), F32),
            pltpu.VMEM((EPB, 24), F32),
            pltpu.VMEM((EPB, 24), F32),
            pltpu.VMEM((NROW_CH, 24), F32),
            pltpu.SemaphoreType.DMA,
            pltpu.SemaphoreType.DMA,
            pltpu.SemaphoreType.DMA,
            pltpu.SemaphoreType.DMA,
            pltpu.VMEM_SHARED((hp, 24), F32),
        ],
    )
    def k(ei_hbm, h2a_hbm, ad_hbm, ap_hbm,
          sd_v, li_v, rh0, rh1, rd0, rd1, oa0, oa1,
          ch_v, sh0, sh1, sdm0, sdm1, acc_sh):
        cid = lax.axis_index("c")
        sid = lax.axis_index("s")
        lane = lax.iota(I32, 16)
        row0 = sid * rows_per_tile
        base = cid * half
        rh = (rh0, rh1)
        rd = (rd0, rd1)
        oa = (oa0, oa1)
        sems_h = (sh0, sh1)
        sems_d = (sdm0, sdm1)

        def _zb(i, _):
            def _zc(j, _):
                plsc.store_scatter(ch_v, [jnp.full((16,), i, I32),
                                          (j * 16 + lane) % 24],
                                   jnp.zeros((16,), F32))
                return 0
            lax.fori_loop(0, 2, _zc, 0)
            return 0
        lax.fori_loop(0, NROW_CH, _zb, 0)

        def _zs(kk, _):
            pltpu.sync_copy(ch_v, acc_sh.at[pl.ds(row0 + kk * NROW_CH,
                                                  NROW_CH)])
            return 0
        lax.fori_loop(0, nch, _zs, 0)

        def _zp(j, _):
            r = j * 16 + lane
            zz = jnp.zeros((16,), F32)
            for col in range(17, 24):
                plsc.store_scatter(oa0, [r, _c16(col)], zz)
                plsc.store_scatter(oa1, [r, _c16(col)], zz)
            return 0
        lax.fori_loop(0, EPB // 16, _zp, 0)

        plsc.subcore_barrier()

        def _prefetch(t, b):
            @pl.when(t * NSUB + sid < nblk)
            def _():
                bs = (t * NSUB + sid) * EPB
                pltpu.sync_copy(ei_hbm.at[:, pl.ds(bs, EPB)], sd_v.at[b])
                pltpu.async_copy(h2a_hbm.at[sd_v.at[b, 0]], rh[b], sems_h[b])
                pltpu.async_copy(ad_hbm.at[sd_v.at[b, 1]], rd[b], sems_d[b])

        def _process(t, b):
            @pl.when(t * NSUB + sid < nblk)
            def _():
                pltpu.make_async_copy(h2a_hbm.at[sd_v.at[b, 0]], rh[b],
                                      sems_h[b]).wait()
                pltpu.make_async_copy(ad_hbm.at[sd_v.at[b, 1]], rd[b],
                                      sems_d[b]).wait()

                def _cmp(j, _):
                    r = j * 16 + lane
                    a_s = plsc.load_gather(rh[b], [r, _c16(16)])
                    a_d = plsc.load_gather(rd[b], [r, _c16(1)])
                    e = a_s + a_d
                    e = jnp.where(e >= 0.0, e, e * 0.2)
                    xv = jnp.exp(e)
                    for d in range(16):
                        v = plsc.load_gather(rh[b], [r, _c16(d)])
                        plsc.store_scatter(oa[b], [r, _c16(d)], v * xv)
                    plsc.store_scatter(oa[b], [r, _c16(16)], xv)
                    dsts = plsc.load_gather(sd_v.at[b, 1], [r])
                    du = dsts - base
                    inr = (du >= 0) & (du < half)
                    li = jnp.where(inr, du, half)
                    li_v[b, pl.ds(j * 16, 16)] = li
                    return 0
                lax.fori_loop(0, EPB // 16, _cmp, 0)

                pltpu.sync_copy(oa[b], acc_sh.at[li_v.at[b]], add=True)

        _prefetch(0, 0)
        tmax = -(-nblk // NSUB)

        def _pair(p, _):
            for b in range(2):
                t = 2 * p + b
                _prefetch(t + 1, 1 - b)
                _process(t, b)
            return 0
        lax.fori_loop(0, (tmax + 1) // 2, _pair, 0)

        plsc.subcore_barrier()
        _flush_spmem_stripe(ch_v, acc_sh, ap_hbm, cid, row0, nch)

    return k, hp


_TC_BLK = 2048


def _tc_prep1(n_nodes):
    def body(x_ref, was_ref, wad_ref, xa_ref, ar_ref):
        xb = x_ref[...]
        a_s = jnp.dot(xb, was_ref[...], preferred_element_type=F32)
        a_d = jnp.dot(xb, wad_ref[...], preferred_element_type=F32)
        z1 = jnp.zeros((xb.shape[0], 1), F32)
        z6 = jnp.zeros((xb.shape[0], 6), F32)
        xa_ref[...] = jnp.concatenate([xb, a_s, z1], axis=1)
        ar_ref[...] = jnp.concatenate([a_d, z6], axis=1)

    return pl.pallas_call(
        body,
        grid=(n_nodes // _TC_BLK,),
        in_specs=[
            pl.BlockSpec((_TC_BLK, 5), lambda i: (i, 0)),
            pl.BlockSpec((5, 2), lambda i: (0, 0)),
            pl.BlockSpec((5, 2), lambda i: (0, 0)),
        ],
        out_specs=[
            pl.BlockSpec((_TC_BLK, 8), lambda i: (i, 0)),
            pl.BlockSpec((_TC_BLK, 8), lambda i: (i, 0)),
        ],
        out_shape=[
            jax.ShapeDtypeStruct((n_nodes, 8), F32),
            jax.ShapeDtypeStruct((n_nodes, 8), F32),
        ],
    )


def _tc_mid(n_nodes):
    def body(gp_ref, w1_ref, b1_ref, w2_ref, att2_ref, h2_ref, ar_ref):
        g = gp_ref[0] + gp_ref[1]
        d0 = g[:, 10:11] + 1e-16
        d1 = g[:, 11:12] + 1e-16
        w1 = w1_ref[...]
        h0 = jnp.dot(g[:, 0:5], w1[:, :32], preferred_element_type=F32) / d0
        h1 = jnp.dot(g[:, 5:10], w1[:, 32:], preferred_element_type=F32) / d1
        h2in = jnp.maximum(jnp.concatenate([h0, h1], axis=1) + b1_ref[...], 0.0)
        h2 = jnp.dot(h2in, w2_ref[...], preferred_element_type=F32)
        att2 = att2_ref[...]
        a_s = jnp.sum(h2 * att2[0:1, :], axis=1, keepdims=True)
        a_d = jnp.sum(h2 * att2[1:2, :], axis=1, keepdims=True)
        z6 = jnp.zeros((h2.shape[0], 6), F32)
        h2_ref[...] = jnp.concatenate([h2, a_s, a_d, z6], axis=1)
        ar_ref[...] = jnp.concatenate([a_s, a_d, z6], axis=1)

    return pl.pallas_call(
        body,
        grid=(n_nodes // _TC_BLK,),
        in_specs=[
            pl.BlockSpec((2, _TC_BLK, 16), lambda i: (0, i, 0)),
            pl.BlockSpec((5, 64), lambda i: (0, 0)),
            pl.BlockSpec((1, 64), lambda i: (0, 0)),
            pl.BlockSpec((64, 16), lambda i: (0, 0)),
            pl.BlockSpec((2, 16), lambda i: (0, 0)),
        ],
        out_specs=[
            pl.BlockSpec((_TC_BLK, 24), lambda i: (i, 0)),
            pl.BlockSpec((_TC_BLK, 8), lambda i: (i, 0)),
        ],
        out_shape=[
            jax.ShapeDtypeStruct((n_nodes, 24), F32),
            jax.ShapeDtypeStruct((n_nodes, 8), F32),
        ],
    )


def _tc_final(n_nodes):
    def body(acc_ref, b2_ref, out_ref):
        acc = acc_ref[...]
        den = acc[:, 16:17] + 1e-16
        out_ref[...] = acc[:, 0:16] / den + b2_ref[...]

    return pl.pallas_call(
        body,
        grid=(n_nodes // _TC_BLK,),
        in_specs=[
            pl.BlockSpec((_TC_BLK, 24), lambda i: (i, 0)),
            pl.BlockSpec((1, 16), lambda i: (0, 0)),
        ],
        out_specs=pl.BlockSpec((_TC_BLK, 16), lambda i: (i, 0)),
        out_shape=jax.ShapeDtypeStruct((n_nodes, 16), F32),
    )


@jax.jit
def kernel(x, edge_index, W1, att_src1, att_dst1, b1, W2, att_src2, att_dst2, b2):
    n_nodes = x.shape[0]
    n_edges = edge_index.shape[1]
    n_pad = -(-n_nodes // (NSUB * NROW_CH)) * (NSUB * NROW_CH)
    ei = edge_index.astype(I32)
    xp = jnp.pad(x, ((0, n_pad - n_nodes), (0, 0)))

    w3 = W1.reshape(x.shape[1], att_src1.shape[0], att_src1.shape[1])
    w_as1 = (w3 * att_src1[None]).sum(-1)
    w_ad1 = (w3 * att_dst1[None]).sum(-1)

    xa, arow1 = _tc_prep1(n_pad)(xp, w_as1, w_ad1)
    gp = _sc_layer1(n_pad, n_edges)(ei, xa, arow1)
    att2 = jnp.concatenate([att_src2, att_dst2], axis=0)
    h2row, arow2 = _tc_mid(n_pad)(gp, W1, b1.reshape(1, -1), W2, att2)
    l2, _hp = _sc_layer2(n_pad, n_edges)
    accp = l2(ei, h2row, arow2)
    half = n_pad // 2
    acc_full = jnp.concatenate([accp[0, :half], accp[1, :half]], axis=0)
    out = _tc_final(n_pad)(acc_full, b2.reshape(1, -1))
    return out[:n_nodes]


# speedup vs baseline: 113.0107x; 1.0962x over previous
import functools

import jax
import jax.numpy as jnp
from jax import lax
from jax.experimental import pallas as pl
from jax.experimental.pallas import tpu as pltpu
from jax.experimental.pallas import tpu_sc as plsc

F32 = jnp.float32
I32 = jnp.int32

EPB = 128
NROW_CH = 128
NW = 32
NSUB = 16

_SC_PARAMS = pltpu.CompilerParams(needs_layout_passes=False,
                                  use_tc_tiling_on_sc=False)


def _mesh():
    return plsc.VectorSubcoreMesh(core_axis_name="c", subcore_axis_name="s",
                                  num_cores=2, num_subcores=NSUB)


def _c16(v):
    return jnp.full((16,), v, dtype=I32)


def _zero_spmem_stripe(ch_v, sh_ref, row0, nch):
    def _zb(i, _):
        ch_v[i, :] = jnp.zeros((16,), F32)
        return 0
    lax.fori_loop(0, NROW_CH, _zb, 0)

    def _zs(kk, _):
        pltpu.sync_copy(ch_v, sh_ref.at[pl.ds(row0 + kk * NROW_CH, NROW_CH)])
        return 0
    lax.fori_loop(0, nch, _zs, 0)


def _flush_spmem_stripe(ch_v, sh_ref, out_hbm, cid, row0, nch):
    def _fl(kk, _):
        r0 = row0 + kk * NROW_CH
        pltpu.sync_copy(sh_ref.at[pl.ds(r0, NROW_CH)], ch_v)
        pltpu.sync_copy(ch_v, out_hbm.at[cid].at[pl.ds(r0, NROW_CH)])
        return 0
    lax.fori_loop(0, nch, _fl, 0)


def _sc_layer1(n_nodes, n_edges):
    nblk = n_edges // EPB
    rows_per_tile = n_nodes // NSUB
    nch = rows_per_tile // NROW_CH
    tmax = -(-nblk // NW)

    @functools.partial(
        pl.kernel,
        out_type=jax.ShapeDtypeStruct((2, n_nodes, 16), F32),
        mesh=_mesh(),
        compiler_params=_SC_PARAMS,
        scratch_types=[
            pltpu.VMEM((2, 2, EPB), I32),
            pltpu.VMEM((2, EPB), I32),
            pltpu.VMEM((EPB, 8), F32),
            pltpu.VMEM((EPB, 8), F32),
            pltpu.VMEM((EPB, 8), F32),
            pltpu.VMEM((EPB, 8), F32),
            pltpu.VMEM((EPB, 16), F32),
            pltpu.VMEM((EPB, 16), F32),
            pltpu.VMEM((NROW_CH, 16), F32),
            pltpu.SemaphoreType.DMA,
            pltpu.SemaphoreType.DMA,
            pltpu.SemaphoreType.DMA,
            pltpu.SemaphoreType.DMA,
            pltpu.SemaphoreType.DMA,
            pltpu.SemaphoreType.DMA,
            pltpu.VMEM_SHARED((n_nodes, 16), F32),
        ],
    )
    def k(ei_hbm, xa_hbm, ad_hbm, gp_hbm,
          sd_v, li_v, rs0, rs1, rd0, rd1, or0, or1, ch_v,
          ss0, ss1, sdm0, sdm1, sc0, sc1, g_sh):
        cid = lax.axis_index("c")
        sid = lax.axis_index("s")
        wid = cid * NSUB + sid
        lane = lax.iota(I32, 16)
        row0 = sid * rows_per_tile
        rs = (rs0, rs1)
        rd = (rd0, rd1)
        orow = (or0, or1)
        sems_s = (ss0, ss1)
        sems_d = (sdm0, sdm1)
        sems_c = (sc0, sc1)

        _zero_spmem_stripe(ch_v, g_sh, row0, nch)

        def _zp(j, _):
            r = j * 16 + lane
            zz = jnp.zeros((16,), F32)
            for col in (12, 13, 14, 15):
                plsc.store_scatter(or0, [r, _c16(col)], zz)
                plsc.store_scatter(or1, [r, _c16(col)], zz)
            return 0
        lax.fori_loop(0, EPB // 16, _zp, 0)

        plsc.subcore_barrier()

        def _prefetch(t, b):
            @pl.when(t * NW + wid < nblk)
            def _():
                base = (t * NW + wid) * EPB
                pltpu.sync_copy(ei_hbm.at[:, pl.ds(base, EPB)], sd_v.at[b])
                pltpu.async_copy(xa_hbm.at[sd_v.at[b, 0]], rs[b], sems_s[b])
                pltpu.async_copy(ad_hbm.at[sd_v.at[b, 1]], rd[b], sems_d[b])

        def _process(t, b):
            @pl.when(t * NW + wid < nblk)
            def _():
                @pl.when(t >= 2)
                def _w():
                    pltpu.make_async_copy(orow[b], g_sh.at[li_v.at[b]],
                                          sems_c[b]).wait()
                pltpu.make_async_copy(xa_hbm.at[sd_v.at[b, 0]], rs[b],
                                      sems_s[b]).wait()
                pltpu.make_async_copy(ad_hbm.at[sd_v.at[b, 1]], rd[b],
                                      sems_d[b]).wait()

                def _cmp(j, _):
                    r = j * 16 + lane
                    as0 = plsc.load_gather(rs[b], [r, _c16(5)])
                    as1 = plsc.load_gather(rs[b], [r, _c16(6)])
                    ad0 = plsc.load_gather(rd[b], [r, _c16(0)])
                    ad1 = plsc.load_gather(rd[b], [r, _c16(1)])
                    e0 = as0 + ad0
                    e0 = jnp.where(e0 >= 0.0, e0, e0 * 0.2)
                    x0 = jnp.exp(e0)
                    e1 = as1 + ad1
                    e1 = jnp.where(e1 >= 0.0, e1, e1 * 0.2)
                    x1 = jnp.exp(e1)
                    for d in range(5):
                        xd = plsc.load_gather(rs[b], [r, _c16(d)])
                        plsc.store_scatter(orow[b], [r, _c16(d)], xd * x0)
                        plsc.store_scatter(orow[b], [r, _c16(5 + d)], xd * x1)
                    plsc.store_scatter(orow[b], [r, _c16(10)], x0)
                    plsc.store_scatter(orow[b], [r, _c16(11)], x1)
                    dsts = plsc.load_gather(sd_v.at[b, 1], [r])
                    li_v[b, pl.ds(j * 16, 16)] = dsts
                    return 0
                lax.fori_loop(0, EPB // 16, _cmp, 0)

                pltpu.async_copy(orow[b], g_sh.at[li_v.at[b]],
                                 sems_c[b], add=True)

        _prefetch(0, 0)
        tlim = 2 * ((tmax + 1) // 2)

        def _pair(p, _):
            for b in range(2):
                t = 2 * p + b
                _prefetch(t + 1, 1 - b)
                _process(t, b)
            return 0
        lax.fori_loop(0, tlim // 2, _pair, 0)

        for tt in (tlim - 2, tlim - 1):
            bb = tt & 1

            @pl.when(tt * NW + wid < nblk)
            def _dr(bb=bb):
                pltpu.make_async_copy(orow[bb], g_sh.at[li_v.at[bb]],
                                      sems_c[bb]).wait()

        plsc.subcore_barrier()
        _flush_spmem_stripe(ch_v, g_sh, gp_hbm, cid, row0, nch)

    return k


def _sc_layer2(n_nodes, n_edges):
    nblk = n_edges // EPB
    half = n_nodes // 2
    hp = ((half // (NSUB * NROW_CH)) + 1) * (NSUB * NROW_CH)
    rows_per_tile = hp // NSUB
    nch = rows_per_tile // NROW_CH

    @functools.partial(
        pl.kernel,
        out_type=jax.ShapeDtypeStruct((2, hp, 24), F32),
        mesh=_mesh(),
        compiler_params=_SC_PARAMS,
        scratch_types=[
            pltpu.VMEM((2, 2, EPB), I32),
            pltpu.VMEM((2, EPB), I32),
            pltpu.VMEM((EPB, 24), F32),
            pltpu.VMEM((EPB, 24), F32),
            pltpu.VMEM((EPB, 8), F32),
            pltpu.VMEM((EPB, 8), F32),
            pltpu.VMEM((EPB, 24), F32),
            pltpu.VMEM((EPB, 24), F32),
            pltpu.VMEM((NROW_CH, 24), F32),
            pltpu.SemaphoreType.DMA,
            pltpu.SemaphoreType.DMA,
            pltpu.SemaphoreType.DMA,
            pltpu.SemaphoreType.DMA,
            pltpu.SemaphoreType.DMA,
            pltpu.SemaphoreType.DMA,
            pltpu.VMEM_SHARED((hp, 24), F32),
        ],
    )
    def k(ei_hbm, h2a_hbm, ad_hbm, ap_hbm,
          sd_v, li_v, rh0, rh1, rd0, rd1, oa0, oa1,
          ch_v, sh0, sh1, sdm0, sdm1, sc0, sc1, acc_sh):
        cid = lax.axis_index("c")
        sid = lax.axis_index("s")
        lane = lax.iota(I32, 16)
        row0 = sid * rows_per_tile
        base = cid * half
        rh = (rh0, rh1)
        rd = (rd0, rd1)
        oa = (oa0, oa1)
        sems_h = (sh0, sh1)
        sems_d = (sdm0, sdm1)
        sems_c = (sc0, sc1)

        def _zb(i, _):
            def _zc(j, _):
                plsc.store_scatter(ch_v, [jnp.full((16,), i, I32),
                                          (j * 16 + lane) % 24],
                                   jnp.zeros((16,), F32))
                return 0
            lax.fori_loop(0, 2, _zc, 0)
            return 0
        lax.fori_loop(0, NROW_CH, _zb, 0)

        def _zs(kk, _):
            pltpu.sync_copy(ch_v, acc_sh.at[pl.ds(row0 + kk * NROW_CH,
                                                  NROW_CH)])
            return 0
        lax.fori_loop(0, nch, _zs, 0)

        def _zp(j, _):
            r = j * 16 + lane
            zz = jnp.zeros((16,), F32)
            for col in range(17, 24):
                plsc.store_scatter(oa0, [r, _c16(col)], zz)
                plsc.store_scatter(oa1, [r, _c16(col)], zz)
            return 0
        lax.fori_loop(0, EPB // 16, _zp, 0)

        plsc.subcore_barrier()

        def _prefetch(t, b):
            @pl.when(t * NSUB + sid < nblk)
            def _():
                bs = (t * NSUB + sid) * EPB
                pltpu.sync_copy(ei_hbm.at[:, pl.ds(bs, EPB)], sd_v.at[b])
                pltpu.async_copy(h2a_hbm.at[sd_v.at[b, 0]], rh[b], sems_h[b])
                pltpu.async_copy(ad_hbm.at[sd_v.at[b, 1]], rd[b], sems_d[b])

        def _process(t, b):
            @pl.when(t * NSUB + sid < nblk)
            def _():
                @pl.when(t >= 2)
                def _w():
                    pltpu.make_async_copy(oa[b], acc_sh.at[li_v.at[b]],
                                          sems_c[b]).wait()
                pltpu.make_async_copy(h2a_hbm.at[sd_v.at[b, 0]], rh[b],
                                      sems_h[b]).wait()
                pltpu.make_async_copy(ad_hbm.at[sd_v.at[b, 1]], rd[b],
                                      sems_d[b]).wait()

                def _cmp(j, _):
                    r = j * 16 + lane
                    a_s = plsc.load_gather(rh[b], [r, _c16(16)])
                    a_d = plsc.load_gather(rd[b], [r, _c16(1)])
                    e = a_s + a_d
                    e = jnp.where(e >= 0.0, e, e * 0.2)
                    xv = jnp.exp(e)
                    for d in range(16):
                        v = plsc.load_gather(rh[b], [r, _c16(d)])
                        plsc.store_scatter(oa[b], [r, _c16(d)], v * xv)
                    plsc.store_scatter(oa[b], [r, _c16(16)], xv)
                    dsts = plsc.load_gather(sd_v.at[b, 1], [r])
                    du = dsts - base
                    inr = (du >= 0) & (du < half)
                    li = jnp.where(inr, du, half)
                    li_v[b, pl.ds(j * 16, 16)] = li
                    return 0
                lax.fori_loop(0, EPB // 16, _cmp, 0)

                pltpu.async_copy(oa[b], acc_sh.at[li_v.at[b]],
                                 sems_c[b], add=True)

        _prefetch(0, 0)
        tmax = -(-nblk // NSUB)
        tlim = 2 * ((tmax + 1) // 2)

        def _pair(p, _):
            for b in range(2):
                t = 2 * p + b
                _prefetch(t + 1, 1 - b)
                _process(t, b)
            return 0
        lax.fori_loop(0, tlim // 2, _pair, 0)

        for tt in (tlim - 2, tlim - 1):
            bb = tt & 1

            @pl.when(tt * NSUB + sid < nblk)
            def _dr(bb=bb):
                pltpu.make_async_copy(oa[bb], acc_sh.at[li_v.at[bb]],
                                      sems_c[bb]).wait()

        plsc.subcore_barrier()
        _flush_spmem_stripe(ch_v, acc_sh, ap_hbm, cid, row0, nch)

    return k, hp


_TC_BLK = 2048


def _tc_prep1(n_nodes):
    def body(x_ref, was_ref, wad_ref, xa_ref, ar_ref):
        xb = x_ref[...]
        a_s = jnp.dot(xb, was_ref[...], preferred_element_type=F32)
        a_d = jnp.dot(xb, wad_ref[...], preferred_element_type=F32)
        z1 = jnp.zeros((xb.shape[0], 1), F32)
        z6 = jnp.zeros((xb.shape[0], 6), F32)
        xa_ref[...] = jnp.concatenate([xb, a_s, z1], axis=1)
        ar_ref[...] = jnp.concatenate([a_d, z6], axis=1)

    return pl.pallas_call(
        body,
        grid=(n_nodes // _TC_BLK,),
        in_specs=[
            pl.BlockSpec((_TC_BLK, 5), lambda i: (i, 0)),
            pl.BlockSpec((5, 2), lambda i: (0, 0)),
            pl.BlockSpec((5, 2), lambda i: (0, 0)),
        ],
        out_specs=[
            pl.BlockSpec((_TC_BLK, 8), lambda i: (i, 0)),
            pl.BlockSpec((_TC_BLK, 8), lambda i: (i, 0)),
        ],
        out_shape=[
            jax.ShapeDtypeStruct((n_nodes, 8), F32),
            jax.ShapeDtypeStruct((n_nodes, 8), F32),
        ],
    )


def _tc_mid(n_nodes):
    def body(gp_ref, w1_ref, b1_ref, w2_ref, att2_ref, h2_ref, ar_ref):
        g = gp_ref[0] + gp_ref[1]
        d0 = g[:, 10:11] + 1e-16
        d1 = g[:, 11:12] + 1e-16
        w1 = w1_ref[...]
        h0 = jnp.dot(g[:, 0:5], w1[:, :32], preferred_element_type=F32) / d0
        h1 = jnp.dot(g[:, 5:10], w1[:, 32:], preferred_element_type=F32) / d1
        h2in = jnp.maximum(jnp.concatenate([h0, h1], axis=1) + b1_ref[...], 0.0)
        h2 = jnp.dot(h2in, w2_ref[...], preferred_element_type=F32)
        att2 = att2_ref[...]
        a_s = jnp.sum(h2 * att2[0:1, :], axis=1, keepdims=True)
        a_d = jnp.sum(h2 * att2[1:2, :], axis=1, keepdims=True)
        z6 = jnp.zeros((h2.shape[0], 6), F32)
        h2_ref[...] = jnp.concatenate([h2, a_s, a_d, z6], axis=1)
        ar_ref[...] = jnp.concatenate([a_s, a_d, z6], axis=1)

    return pl.pallas_call(
        body,
        grid=(n_nodes // _TC_BLK,),
        in_specs=[
            pl.BlockSpec((2, _TC_BLK, 16), lambda i: (0, i, 0)),
            pl.BlockSpec((5, 64), lambda i: (0, 0)),
            pl.BlockSpec((1, 64), lambda i: (0, 0)),
            pl.BlockSpec((64, 16), lambda i: (0, 0)),
            pl.BlockSpec((2, 16), lambda i: (0, 0)),
        ],
        out_specs=[
            pl.BlockSpec((_TC_BLK, 24), lambda i: (i, 0)),
            pl.BlockSpec((_TC_BLK, 8), lambda i: (i, 0)),
        ],
        out_shape=[
            jax.ShapeDtypeStruct((n_nodes, 24), F32),
            jax.ShapeDtypeStruct((n_nodes, 8), F32),
        ],
    )


def _tc_final(n_nodes, hp):
    blk = 1024
    half = n_nodes // 2
    nb_half = half // blk

    def body(acc_ref, b2_ref, out_ref):
        acc = acc_ref[0]
        den = acc[:, 16:17] + 1e-16
        out_ref[...] = acc[:, 0:16] / den + b2_ref[...]

    return pl.pallas_call(
        body,
        grid=(n_nodes // blk,),
        in_specs=[
            pl.BlockSpec((1, blk, 24),
                         lambda i: (i // nb_half, i % nb_half, 0)),
            pl.BlockSpec((1, 16), lambda i: (0, 0)),
        ],
        out_specs=pl.BlockSpec((blk, 16), lambda i: (i, 0)),
        out_shape=jax.ShapeDtypeStruct((n_nodes, 16), F32),
    )


@jax.jit
def kernel(x, edge_index, W1, att_src1, att_dst1, b1, W2, att_src2, att_dst2, b2):
    n_nodes = x.shape[0]
    n_edges = edge_index.shape[1]
    n_pad = -(-n_nodes // (NSUB * NROW_CH)) * (NSUB * NROW_CH)
    ei = edge_index.astype(I32)
    xp = jnp.pad(x, ((0, n_pad - n_nodes), (0, 0)))

    w3 = W1.reshape(x.shape[1], att_src1.shape[0], att_src1.shape[1])
    w_as1 = (w3 * att_src1[None]).sum(-1)
    w_ad1 = (w3 * att_dst1[None]).sum(-1)

    xa, arow1 = _tc_prep1(n_pad)(xp, w_as1, w_ad1)
    gp = _sc_layer1(n_pad, n_edges)(ei, xa, arow1)
    att2 = jnp.concatenate([att_src2, att_dst2], axis=0)
    h2row, arow2 = _tc_mid(n_pad)(gp, W1, b1.reshape(1, -1), W2, att2)
    l2, hp = _sc_layer2(n_pad, n_edges)
    accp = l2(ei, h2row, arow2)
    out = _tc_final(n_pad, hp)(accp, b2.reshape(1, -1))
    return out[:n_nodes]


# speedup vs baseline: 128.2462x; 1.1348x over previous
import functools

import jax
import jax.numpy as jnp
from jax import lax
from jax.experimental import pallas as pl
from jax.experimental.pallas import tpu as pltpu
from jax.experimental.pallas import tpu_sc as plsc

F32 = jnp.float32
I32 = jnp.int32

EPB = 256
SCCH = 128
NROW_CH = 128
NW = 32
NSUB = 16

_SC_PARAMS = pltpu.CompilerParams(needs_layout_passes=False,
                                  use_tc_tiling_on_sc=False)


def _mesh():
    return plsc.VectorSubcoreMesh(core_axis_name="c", subcore_axis_name="s",
                                  num_cores=2, num_subcores=NSUB)


def _c16(v):
    return jnp.full((16,), v, dtype=I32)


def _zero_spmem_stripe(ch_v, sh_ref, row0, nch):
    def _zb(i, _):
        ch_v[i, :] = jnp.zeros((16,), F32)
        return 0
    lax.fori_loop(0, NROW_CH, _zb, 0)

    def _zs(kk, _):
        pltpu.sync_copy(ch_v, sh_ref.at[pl.ds(row0 + kk * NROW_CH, NROW_CH)])
        return 0
    lax.fori_loop(0, nch, _zs, 0)


def _flush_spmem_stripe(ch_v, sh_ref, out_hbm, cid, row0, nch):
    def _fl(kk, _):
        r0 = row0 + kk * NROW_CH
        pltpu.sync_copy(sh_ref.at[pl.ds(r0, NROW_CH)], ch_v)
        pltpu.sync_copy(ch_v, out_hbm.at[cid].at[pl.ds(r0, NROW_CH)])
        return 0
    lax.fori_loop(0, nch, _fl, 0)


def _sc_layer1(n_nodes, n_edges):
    nblk = n_edges // EPB
    rows_per_tile = n_nodes // NSUB
    nch = rows_per_tile // NROW_CH
    tmax = -(-nblk // NW)

    @functools.partial(
        pl.kernel,
        out_type=jax.ShapeDtypeStruct((2, n_nodes, 16), F32),
        mesh=_mesh(),
        compiler_params=_SC_PARAMS,
        scratch_types=[
            pltpu.VMEM((2, 2, EPB), I32),
            pltpu.VMEM((2, 2, SCCH), I32),
            pltpu.VMEM((EPB, 8), F32),
            pltpu.VMEM((EPB, 8), F32),
            pltpu.VMEM((EPB, 8), F32),
            pltpu.VMEM((EPB, 8), F32),
            pltpu.VMEM((EPB, 16), F32),
            pltpu.VMEM((EPB, 16), F32),
            pltpu.VMEM((NROW_CH, 16), F32),
            pltpu.SemaphoreType.DMA,
            pltpu.SemaphoreType.DMA,
            pltpu.SemaphoreType.DMA,
            pltpu.SemaphoreType.DMA,
            pltpu.SemaphoreType.DMA,
            pltpu.SemaphoreType.DMA,
            pltpu.VMEM_SHARED((n_nodes, 16), F32),
        ],
    )
    def k(ei_hbm, xa_hbm, ad_hbm, gp_hbm,
          sd_v, li_v, rs0, rs1, rd0, rd1, or0, or1, ch_v,
          ss0, ss1, sdm0, sdm1, sc0, sc1, g_sh):
        cid = lax.axis_index("c")
        sid = lax.axis_index("s")
        wid = cid * NSUB + sid
        lane = lax.iota(I32, 16)
        row0 = sid * rows_per_tile
        rs = (rs0, rs1)
        rd = (rd0, rd1)
        orow = (or0, or1)
        sems_s = (ss0, ss1)
        sems_d = (sdm0, sdm1)
        sems_c = (sc0, sc1)

        _zero_spmem_stripe(ch_v, g_sh, row0, nch)

        def _zp(j, _):
            r = j * 16 + lane
            zz = jnp.zeros((16,), F32)
            for col in (12, 13, 14, 15):
                plsc.store_scatter(or0, [r, _c16(col)], zz)
                plsc.store_scatter(or1, [r, _c16(col)], zz)
            return 0
        lax.fori_loop(0, EPB // 16, _zp, 0)

        plsc.subcore_barrier()

        def _prefetch(t, b):
            @pl.when(t * NW + wid < nblk)
            def _():
                base = (t * NW + wid) * EPB
                pltpu.sync_copy(ei_hbm.at[:, pl.ds(base, EPB)], sd_v.at[b])
                pltpu.async_copy(xa_hbm.at[sd_v.at[b, 0]], rs[b], sems_s[b])
                pltpu.async_copy(ad_hbm.at[sd_v.at[b, 1]], rd[b], sems_d[b])

        def _process(t, b):
            @pl.when(t * NW + wid < nblk)
            def _():
                @pl.when(t >= 2)
                def _w():
                    for hh in range(EPB // SCCH):
                        pltpu.make_async_copy(
                            orow[b].at[pl.ds(hh * SCCH, SCCH)],
                            g_sh.at[li_v.at[b, hh]], sems_c[b]).wait()
                pltpu.make_async_copy(xa_hbm.at[sd_v.at[b, 0]], rs[b],
                                      sems_s[b]).wait()
                pltpu.make_async_copy(ad_hbm.at[sd_v.at[b, 1]], rd[b],
                                      sems_d[b]).wait()

                def _cmp(j, hh, _):
                    r = j * 16 + lane
                    as0 = plsc.load_gather(rs[b], [r, _c16(5)])
                    as1 = plsc.load_gather(rs[b], [r, _c16(6)])
                    ad0 = plsc.load_gather(rd[b], [r, _c16(0)])
                    ad1 = plsc.load_gather(rd[b], [r, _c16(1)])
                    e0 = as0 + ad0
                    e0 = jnp.where(e0 >= 0.0, e0, e0 * 0.2)
                    x0 = jnp.exp(e0)
                    e1 = as1 + ad1
                    e1 = jnp.where(e1 >= 0.0, e1, e1 * 0.2)
                    x1 = jnp.exp(e1)
                    for d in range(5):
                        xd = plsc.load_gather(rs[b], [r, _c16(d)])
                        plsc.store_scatter(orow[b], [r, _c16(d)], xd * x0)
                        plsc.store_scatter(orow[b], [r, _c16(5 + d)], xd * x1)
                    plsc.store_scatter(orow[b], [r, _c16(10)], x0)
                    plsc.store_scatter(orow[b], [r, _c16(11)], x1)
                    dsts = plsc.load_gather(sd_v.at[b, 1], [r])
                    li_v[b, hh, pl.ds((j - hh * (SCCH // 16)) * 16, 16)] = dsts
                    return 0
                for hh in range(EPB // SCCH):
                    lax.fori_loop(hh * (SCCH // 16), (hh + 1) * (SCCH // 16),
                                  lambda j, c, hh=hh: _cmp(j, hh, c), 0)
                    pltpu.async_copy(orow[b].at[pl.ds(hh * SCCH, SCCH)],
                                     g_sh.at[li_v.at[b, hh]],
                                     sems_c[b], add=True)

        _prefetch(0, 0)
        tlim = 2 * ((tmax + 1) // 2)

        def _pair(p, _):
            for b in range(2):
                t = 2 * p + b
                _prefetch(t + 1, 1 - b)
                _process(t, b)
            return 0
        lax.fori_loop(0, tlim // 2, _pair, 0)

        for tt in (tlim - 2, tlim - 1):
            bb = tt & 1

            @pl.when(tt * NW + wid < nblk)
            def _dr(bb=bb):
                for hh in range(EPB // SCCH):
                    pltpu.make_async_copy(
                        orow[bb].at[pl.ds(hh * SCCH, SCCH)],
                        g_sh.at[li_v.at[bb, hh]], sems_c[bb]).wait()

        plsc.subcore_barrier()
        _flush_spmem_stripe(ch_v, g_sh, gp_hbm, cid, row0, nch)

    return k


def _sc_layer2(n_nodes, n_edges):
    nblk = n_edges // EPB
    half = n_nodes // 2
    hp = ((half // (NSUB * NROW_CH)) + 1) * (NSUB * NROW_CH)
    rows_per_tile = hp // NSUB
    nch = rows_per_tile // NROW_CH

    @functools.partial(
        pl.kernel,
        out_type=jax.ShapeDtypeStruct((2, hp, 24), F32),
        mesh=_mesh(),
        compiler_params=_SC_PARAMS,
        scratch_types=[
            pltpu.VMEM((2, 2, EPB), I32),
            pltpu.VMEM((2, 2, SCCH), I32),
            pltpu.VMEM((EPB, 24), F32),
            pltpu.VMEM((EPB, 24), F32),
            pltpu.VMEM((EPB, 8), F32),
            pltpu.VMEM((EPB, 8), F32),
            pltpu.VMEM((EPB, 24), F32),
            pltpu.VMEM((EPB, 24), F32),
            pltpu.VMEM((NROW_CH, 24), F32),
            pltpu.SemaphoreType.DMA,
            pltpu.SemaphoreType.DMA,
            pltpu.SemaphoreType.DMA,
            pltpu.SemaphoreType.DMA,
            pltpu.SemaphoreType.DMA,
            pltpu.SemaphoreType.DMA,
            pltpu.VMEM_SHARED((hp, 24), F32),
        ],
    )
    def k(ei_hbm, h2a_hbm, ad_hbm, ap_hbm,
          sd_v, li_v, rh0, rh1, rd0, rd1, oa0, oa1,
          ch_v, sh0, sh1, sdm0, sdm1, sc0, sc1, acc_sh):
        cid = lax.axis_index("c")
        sid = lax.axis_index("s")
        lane = lax.iota(I32, 16)
        row0 = sid * rows_per_tile
        base = cid * half
        rh = (rh0, rh1)
        rd = (rd0, rd1)
        oa = (oa0, oa1)
        sems_h = (sh0, sh1)
        sems_d = (sdm0, sdm1)
        sems_c = (sc0, sc1)

        def _zb(i, _):
            def _zc(j, _):
                plsc.store_scatter(ch_v, [jnp.full((16,), i, I32),
                                          (j * 16 + lane) % 24],
                                   jnp.zeros((16,), F32))
                return 0
            lax.fori_loop(0, 2, _zc, 0)
            return 0
        lax.fori_loop(0, NROW_CH, _zb, 0)

        def _zs(kk, _):
            pltpu.sync_copy(ch_v, acc_sh.at[pl.ds(row0 + kk * NROW_CH,
                                                  NROW_CH)])
            return 0
        lax.fori_loop(0, nch, _zs, 0)

        def _zp(j, _):
            r = j * 16 + lane
            zz = jnp.zeros((16,), F32)
            for col in range(17, 24):
                plsc.store_scatter(oa0, [r, _c16(col)], zz)
                plsc.store_scatter(oa1, [r, _c16(col)], zz)
            return 0
        lax.fori_loop(0, EPB // 16, _zp, 0)

        plsc.subcore_barrier()

        def _prefetch(t, b):
            @pl.when(t * NSUB + sid < nblk)
            def _():
                bs = (t * NSUB + sid) * EPB
                pltpu.sync_copy(ei_hbm.at[:, pl.ds(bs, EPB)], sd_v.at[b])
                pltpu.async_copy(h2a_hbm.at[sd_v.at[b, 0]], rh[b], sems_h[b])
                pltpu.async_copy(ad_hbm.at[sd_v.at[b, 1]], rd[b], sems_d[b])

        def _process(t, b):
            @pl.when(t * NSUB + sid < nblk)
            def _():
                @pl.when(t >= 2)
                def _w():
                    for hh in range(EPB // SCCH):
                        pltpu.make_async_copy(
                            oa[b].at[pl.ds(hh * SCCH, SCCH)],
                            acc_sh.at[li_v.at[b, hh]], sems_c[b]).wait()
                pltpu.make_async_copy(h2a_hbm.at[sd_v.at[b, 0]], rh[b],
                                      sems_h[b]).wait()
                pltpu.make_async_copy(ad_hbm.at[sd_v.at[b, 1]], rd[b],
                                      sems_d[b]).wait()

                def _cmp(j, hh, _):
                    r = j * 16 + lane
                    a_s = plsc.load_gather(rh[b], [r, _c16(16)])
                    a_d = plsc.load_gather(rd[b], [r, _c16(1)])
                    e = a_s + a_d
                    e = jnp.where(e >= 0.0, e, e * 0.2)
                    xv = jnp.exp(e)
                    for d in range(16):
                        v = plsc.load_gather(rh[b], [r, _c16(d)])
                        plsc.store_scatter(oa[b], [r, _c16(d)], v * xv)
                    plsc.store_scatter(oa[b], [r, _c16(16)], xv)
                    dsts = plsc.load_gather(sd_v.at[b, 1], [r])
                    du = dsts - base
                    inr = (du >= 0) & (du < half)
                    li = jnp.where(inr, du, half)
                    li_v[b, hh, pl.ds((j - hh * (SCCH // 16)) * 16, 16)] = li
                    return 0
                for hh in range(EPB // SCCH):
                    lax.fori_loop(hh * (SCCH // 16), (hh + 1) * (SCCH // 16),
                                  lambda j, c, hh=hh: _cmp(j, hh, c), 0)
                    pltpu.async_copy(oa[b].at[pl.ds(hh * SCCH, SCCH)],
                                     acc_sh.at[li_v.at[b, hh]],
                                     sems_c[b], add=True)

        _prefetch(0, 0)
        tmax = -(-nblk // NSUB)
        tlim = 2 * ((tmax + 1) // 2)

        def _pair(p, _):
            for b in range(2):
                t = 2 * p + b
                _prefetch(t + 1, 1 - b)
                _process(t, b)
            return 0
        lax.fori_loop(0, tlim // 2, _pair, 0)

        for tt in (tlim - 2, tlim - 1):
            bb = tt & 1

            @pl.when(tt * NSUB + sid < nblk)
            def _dr(bb=bb):
                for hh in range(EPB // SCCH):
                    pltpu.make_async_copy(
                        oa[bb].at[pl.ds(hh * SCCH, SCCH)],
                        acc_sh.at[li_v.at[bb, hh]], sems_c[bb]).wait()

        plsc.subcore_barrier()
        _flush_spmem_stripe(ch_v, acc_sh, ap_hbm, cid, row0, nch)

    return k, hp


_TC_BLK = 2048


def _tc_prep1(n_nodes):
    def body(x_ref, was_ref, wad_ref, xa_ref, ar_ref):
        xb = x_ref[...]
        a_s = jnp.dot(xb, was_ref[...], preferred_element_type=F32)
        a_d = jnp.dot(xb, wad_ref[...], preferred_element_type=F32)
        z1 = jnp.zeros((xb.shape[0], 1), F32)
        z6 = jnp.zeros((xb.shape[0], 6), F32)
        xa_ref[...] = jnp.concatenate([xb, a_s, z1], axis=1)
        ar_ref[...] = jnp.concatenate([a_d, z6], axis=1)

    return pl.pallas_call(
        body,
        grid=(n_nodes // _TC_BLK,),
        in_specs=[
            pl.BlockSpec((_TC_BLK, 5), lambda i: (i, 0)),
            pl.BlockSpec((5, 2), lambda i: (0, 0)),
            pl.BlockSpec((5, 2), lambda i: (0, 0)),
        ],
        out_specs=[
            pl.BlockSpec((_TC_BLK, 8), lambda i: (i, 0)),
            pl.BlockSpec((_TC_BLK, 8), lambda i: (i, 0)),
        ],
        out_shape=[
            jax.ShapeDtypeStruct((n_nodes, 8), F32),
            jax.ShapeDtypeStruct((n_nodes, 8), F32),
        ],
    )


def _tc_mid(n_nodes):
    def body(gp_ref, w1_ref, b1_ref, w2_ref, att2_ref, h2_ref, ar_ref):
        g = gp_ref[0] + gp_ref[1]
        d0 = g[:, 10:11] + 1e-16
        d1 = g[:, 11:12] + 1e-16
        w1 = w1_ref[...]
        h0 = jnp.dot(g[:, 0:5], w1[:, :32], preferred_element_type=F32) / d0
        h1 = jnp.dot(g[:, 5:10], w1[:, 32:], preferred_element_type=F32) / d1
        h2in = jnp.maximum(jnp.concatenate([h0, h1], axis=1) + b1_ref[...], 0.0)
        h2 = jnp.dot(h2in, w2_ref[...], preferred_element_type=F32)
        att2 = att2_ref[...]
        a_s = jnp.sum(h2 * att2[0:1, :], axis=1, keepdims=True)
        a_d = jnp.sum(h2 * att2[1:2, :], axis=1, keepdims=True)
        z6 = jnp.zeros((h2.shape[0], 6), F32)
        h2_ref[...] = jnp.concatenate([h2, a_s, a_d, z6], axis=1)
        ar_ref[...] = jnp.concatenate([a_s, a_d, z6], axis=1)

    return pl.pallas_call(
        body,
        grid=(n_nodes // _TC_BLK,),
        in_specs=[
            pl.BlockSpec((2, _TC_BLK, 16), lambda i: (0, i, 0)),
            pl.BlockSpec((5, 64), lambda i: (0, 0)),
            pl.BlockSpec((1, 64), lambda i: (0, 0)),
            pl.BlockSpec((64, 16), lambda i: (0, 0)),
            pl.BlockSpec((2, 16), lambda i: (0, 0)),
        ],
        out_specs=[
            pl.BlockSpec((_TC_BLK, 24), lambda i: (i, 0)),
            pl.BlockSpec((_TC_BLK, 8), lambda i: (i, 0)),
        ],
        out_shape=[
            jax.ShapeDtypeStruct((n_nodes, 24), F32),
            jax.ShapeDtypeStruct((n_nodes, 8), F32),
        ],
    )


def _tc_final(n_nodes, hp):
    blk = 1024
    half = n_nodes // 2
    nb_half = half // blk

    def body(acc_ref, b2_ref, out_ref):
        acc = acc_ref[0]
        den = acc[:, 16:17] + 1e-16
        out_ref[...] = acc[:, 0:16] / den + b2_ref[...]

    return pl.pallas_call(
        body,
        grid=(n_nodes // blk,),
        in_specs=[
            pl.BlockSpec((1, blk, 24),
                         lambda i: (i // nb_half, i % nb_half, 0)),
            pl.BlockSpec((1, 16), lambda i: (0, 0)),
        ],
        out_specs=pl.BlockSpec((blk, 16), lambda i: (i, 0)),
        out_shape=jax.ShapeDtypeStruct((n_nodes, 16), F32),
    )


@jax.jit
def kernel(x, edge_index, W1, att_src1, att_dst1, b1, W2, att_src2, att_dst2, b2):
    n_nodes = x.shape[0]
    n_edges = edge_index.shape[1]
    n_pad = -(-n_nodes // (NSUB * NROW_CH)) * (NSUB * NROW_CH)
    ei = edge_index.astype(I32)
    xp = jnp.pad(x, ((0, n_pad - n_nodes), (0, 0)))

    w3 = W1.reshape(x.shape[1], att_src1.shape[0], att_src1.shape[1])
    w_as1 = (w3 * att_src1[None]).sum(-1)
    w_ad1 = (w3 * att_dst1[None]).sum(-1)

    xa, arow1 = _tc_prep1(n_pad)(xp, w_as1, w_ad1)
    gp = _sc_layer1(n_pad, n_edges)(ei, xa, arow1)
    att2 = jnp.concatenate([att_src2, att_dst2], axis=0)
    h2row, arow2 = _tc_mid(n_pad)(gp, W1, b1.reshape(1, -1), W2, att2)
    l2, hp = _sc_layer2(n_pad, n_edges)
    accp = l2(ei, h2row, arow2)
    out = _tc_final(n_pad, hp)(accp, b2.reshape(1, -1))
    return out[:n_nodes]
